# baseline (speedup 1.0000x reference)
import functools

import jax
import jax.numpy as jnp
from jax import lax
from jax.experimental import pallas as pl
from jax.experimental.pallas import tpu as pltpu
from jax.experimental.pallas import tpu_sc as plsc

LENGTHS = (8, 16, 32, 64, 128, 256)
MIN_SCORE = 0.1
K = 1000
B, T = 64, 8192
G = len(LENGTHS)
C = G * T
BITS_MIN = 1036831949
BITS_ONE = 1065353216
ROW_BLK = 8
CPAD = 1024
PAD_VIDX = 1 << 20


def _score_threshold_body(sp_ref, ep_ref, keys_ref, thr_ref):
    sp = sp_ref[...]
    ep = ep_ref[...]
    parts = []
    for g, length in enumerate(LENGTHS):
        w = T - length + 1
        sc = jnp.sqrt(sp[:, :w] * ep[:, length - 1:])
        kb = lax.bitcast_convert_type(sc, jnp.int32)
        kb = jnp.where(sc >= MIN_SCORE, kb, 0)
        parts.append(kb)
        parts.append(jnp.zeros((ROW_BLK, length - 1), jnp.int32))
    keys = jnp.concatenate(parts, axis=1)
    keys_ref[...] = keys

    def bs_cond(st):
        t, lo, hi, vsel, done = st
        return jnp.logical_and(t < 26, jnp.sum(done) < ROW_BLK)

    def bs_step(st):
        t, lo, hi, vsel, done = st
        mid = (lo + hi) >> 1
        cnt = jnp.sum((keys > mid).astype(jnp.int32), axis=1, keepdims=True)
        small = cnt < K
        doneb = done > 0
        in_win = (cnt >= K) & (cnt <= CPAD) & jnp.logical_not(doneb)
        frozen = doneb | in_win
        lo = jnp.where(frozen, lo, jnp.where(small, lo, mid + 1))
        hi = jnp.where(frozen, hi, jnp.where(small, mid, hi))
        vsel = jnp.where(in_win, mid + 1, vsel)
        conv = (lo >= hi) & jnp.logical_not(frozen)
        vsel = jnp.where(conv, hi, vsel)
        done = jnp.where(in_win | conv, 1, done)
        return t + 1, lo, hi, vsel, done

    lo0 = jnp.full((ROW_BLK, 1), BITS_MIN - 1, jnp.int32)
    hi0 = jnp.full((ROW_BLK, 1), BITS_ONE, jnp.int32)
    done0 = jnp.zeros((ROW_BLK, 1), jnp.int32)
    _, _, hi_f, vsel_f, done_f = lax.while_loop(
        bs_cond, bs_step, (jnp.int32(0), lo0, hi0, hi0, done0))
    thr = jnp.where(done_f > 0, vsel_f, hi_f)
    thr_ref[...] = jnp.broadcast_to(thr, (ROW_BLK, 16))


def _score_threshold(sp, ep):
    return pl.pallas_call(
        _score_threshold_body,
        grid=(B // ROW_BLK,),
        in_specs=[
            pl.BlockSpec((ROW_BLK, T), lambda r: (r, 0)),
            pl.BlockSpec((ROW_BLK, T), lambda r: (r, 0)),
        ],
        out_specs=[
            pl.BlockSpec((ROW_BLK, C), lambda r: (r, 0)),
            pl.BlockSpec((ROW_BLK, 16), lambda r: (r, 0)),
        ],
        out_shape=[
            jax.ShapeDtypeStruct((B, C), jnp.int32),
            jax.ShapeDtypeStruct((B, 16), jnp.int32),
        ],
        compiler_params=pltpu.CompilerParams(
            dimension_semantics=("parallel",)),
    )(sp, ep)


ROWS_PER_TILE = 2


def _compact_body(keys_hbm, thr_hbm, ovidx_hbm, okey_hbm,
                  keys_v, cvidx_v, ckey_v, thr_v):
    nc = 2
    wid = lax.axis_index("s") * nc + lax.axis_index("c")
    iota16 = lax.broadcasted_iota(jnp.int32, (16,), 0)
    for rr in range(ROWS_PER_TILE):
        row = wid * ROWS_PER_TILE + rr
        pltpu.sync_copy(keys_hbm.at[row], keys_v)
        pltpu.sync_copy(thr_hbm.at[row], thr_v)
        vthr = thr_v[...]

        @plsc.parallel_loop(0, CPAD // 16, unroll=4)
        def init_step(j):
            cvidx_v[pl.ds(j * 16, 16)] = PAD_VIDX + j * 16 + iota16
            ckey_v[pl.ds(j * 16, 16)] = jnp.zeros((16,), jnp.int32)

        @plsc.parallel_loop(0, C // 16, unroll=8,
                            carry=jnp.zeros((16,), jnp.int32))
        def scan_step(i, off):
            k = keys_v[pl.ds(i * 16, 16)]
            m = k >= vthr
            pc = plsc.cumsum(jnp.where(m, 1, 0))
            pos = off + pc - 1
            em = m & (pos < CPAD)
            plsc.store_scatter(cvidx_v, [pos], i * 16 + iota16, mask=em)
            plsc.store_scatter(ckey_v, [pos], k, mask=em)
            return off + plsc.all_reduce_population_count(m)

        pltpu.sync_copy(cvidx_v, ovidx_hbm.at[row])
        pltpu.sync_copy(ckey_v, okey_hbm.at[row])


@functools.lru_cache(maxsize=1)
def _make_compact():
    return pl.kernel(
        _compact_body,
        out_type=(jax.ShapeDtypeStruct((B, CPAD), jnp.int32),
                  jax.ShapeDtypeStruct((B, CPAD), jnp.int32)),
        mesh=plsc.VectorSubcoreMesh(core_axis_name="c", subcore_axis_name="s"),
        scratch_types=[
            pltpu.VMEM((C,), jnp.int32),
            pltpu.VMEM((CPAD,), jnp.int32),
            pltpu.VMEM((CPAD,), jnp.int32),
            pltpu.VMEM((16,), jnp.int32),
        ],
        compiler_params=pltpu.CompilerParams(needs_layout_passes=False),
    )


RB3 = 8


def _rank_vals_body(vidx_ref, key_ref, vidxT_ref, keyT_ref, rank_ref, vals_ref):
    for rr in range(RB3):
        kj = key_ref[rr]
        vj = vidx_ref[rr]
        ki = keyT_ref[rr]
        vi = vidxT_ref[rr]
        beats = (kj > ki) | ((kj == ki) & (vj < vi))
        rank = jnp.sum(beats.astype(jnp.int32), axis=1, keepdims=True)
        rank_ref[rr] = rank
        g = jnp.minimum(lax.shift_right_logical(vj, 13), 5)
        length = jnp.left_shift(jnp.int32(8), g)
        s = jnp.bitwise_and(vj, T - 1)
        valid = kj >= BITS_MIN
        sf = jnp.where(valid, s.astype(jnp.float32), 0.0)
        ef = jnp.where(valid, (s + length).astype(jnp.float32), float(T))
        scf = jnp.where(valid, lax.bitcast_convert_type(kj, jnp.float32), 0.0)
        vals_ref[rr] = jnp.concatenate([sf, ef, scf], axis=0)


def _rank_vals(cvidx, ckey, cvidx_t, ckey_t):
    return pl.pallas_call(
        _rank_vals_body,
        grid=(B // RB3,),
        in_specs=[
            pl.BlockSpec((RB3, 1, CPAD), lambda r: (r, 0, 0)),
            pl.BlockSpec((RB3, 1, CPAD), lambda r: (r, 0, 0)),
            pl.BlockSpec((RB3, CPAD, 1), lambda r: (r, 0, 0)),
            pl.BlockSpec((RB3, CPAD, 1), lambda r: (r, 0, 0)),
        ],
        out_specs=[
            pl.BlockSpec((RB3, CPAD, 1), lambda r: (r, 0, 0)),
            pl.BlockSpec((RB3, 3, CPAD), lambda r: (r, 0, 0)),
        ],
        out_shape=[
            jax.ShapeDtypeStruct((B, CPAD, 1), jnp.int32),
            jax.ShapeDtypeStruct((B, 3, CPAD), jnp.float32),
        ],
        compiler_params=pltpu.CompilerParams(
            dimension_semantics=("parallel",)),
    )(cvidx[:, None, :], ckey[:, None, :], cvidx_t, ckey_t)


def _permute_body(rank_hbm, vals_hbm, out_hbm, rank_v, vals_v, out_v):
    nc = 2
    wid = lax.axis_index("s") * nc + lax.axis_index("c")
    iota16 = lax.broadcasted_iota(jnp.int32, (16,), 0)
    del iota16
    for rr in range(ROWS_PER_TILE):
        row = wid * ROWS_PER_TILE + rr
        pltpu.sync_copy(rank_hbm.at[row], rank_v)
        pltpu.sync_copy(vals_hbm.at[row], vals_v)

        @plsc.parallel_loop(0, CPAD // 16, unroll=4)
        def step(i):
            r3 = rank_v[pl.ds(i * 16, 16)] * 3
            for c in range(3):
                x = vals_v[c, pl.ds(i * 16, 16)]
                plsc.store_scatter(out_v, [r3 + c], x)

        pltpu.sync_copy(out_v, out_hbm.at[row])


@functools.lru_cache(maxsize=1)
def _make_permute():
    return pl.kernel(
        _permute_body,
        out_type=jax.ShapeDtypeStruct((B, CPAD * 3), jnp.float32),
        mesh=plsc.VectorSubcoreMesh(core_axis_name="c", subcore_axis_name="s"),
        scratch_types=[
            pltpu.VMEM((CPAD,), jnp.int32),
            pltpu.VMEM((3, CPAD), jnp.float32),
            pltpu.VMEM((CPAD * 3,), jnp.float32),
        ],
        compiler_params=pltpu.CompilerParams(needs_layout_passes=False),
    )


def kernel(start_probs, end_probs):
    sp = start_probs[:, 0, :]
    ep = end_probs[:, 0, :]
    keys, thr = _score_threshold(sp, ep)
    cvidx, ckey = _make_compact()(keys, thr)
    rank, vals = _rank_vals(cvidx, ckey, cvidx[:, :, None], ckey[:, :, None])
    out = _make_permute()(rank[:, :, 0], vals)
    return out.reshape(B, CPAD, 3)[:, :K, :]


# speedup vs baseline: 1.0043x; 1.0043x over previous
import functools

import jax
import jax.numpy as jnp
from jax import lax
from jax.experimental import pallas as pl
from jax.experimental.pallas import tpu as pltpu
from jax.experimental.pallas import tpu_sc as plsc

LENGTHS = (8, 16, 32, 64, 128, 256)
MIN_SCORE = 0.1
K = 1000
B, T = 64, 8192
G = len(LENGTHS)
C = G * T
BITS_MIN = 1036831949
BITS_ONE = 1065353216
ROW_BLK = 8
CPAD = 1024
PAD_VIDX = 1 << 20


def _score_threshold_body(sp_ref, ep_ref, keys_ref, thr_ref):
    sp = sp_ref[...]
    ep = ep_ref[...]
    parts = []
    for g, length in enumerate(LENGTHS):
        w = T - length + 1
        sc = jnp.sqrt(sp[:, :w] * ep[:, length - 1:])
        kb = lax.bitcast_convert_type(sc, jnp.int32)
        kb = jnp.where(sc >= MIN_SCORE, kb, 0)
        parts.append(kb)
        parts.append(jnp.zeros((ROW_BLK, length - 1), jnp.int32))
    keys = jnp.concatenate(parts, axis=1)
    keys_ref[...] = keys

    def bs_cond(st):
        t, lo, hi, vsel, done = st
        return jnp.logical_and(t < 26, jnp.sum(done) < ROW_BLK)

    def bs_step(st):
        t, lo, hi, vsel, done = st
        mid = (lo + hi) >> 1
        cnt = jnp.sum((keys > mid).astype(jnp.int32), axis=1, keepdims=True)
        small = cnt < K
        doneb = done > 0
        in_win = (cnt >= K) & (cnt <= CPAD) & jnp.logical_not(doneb)
        frozen = doneb | in_win
        lo = jnp.where(frozen, lo, jnp.where(small, lo, mid + 1))
        hi = jnp.where(frozen, hi, jnp.where(small, mid, hi))
        vsel = jnp.where(in_win, mid + 1, vsel)
        conv = (lo >= hi) & jnp.logical_not(frozen)
        vsel = jnp.where(conv, hi, vsel)
        done = jnp.where(in_win | conv, 1, done)
        return t + 1, lo, hi, vsel, done

    lo0 = jnp.full((ROW_BLK, 1), BITS_MIN - 1, jnp.int32)
    hi0 = jnp.full((ROW_BLK, 1), BITS_ONE, jnp.int32)
    done0 = jnp.zeros((ROW_BLK, 1), jnp.int32)
    _, _, hi_f, vsel_f, done_f = lax.while_loop(
        bs_cond, bs_step, (jnp.int32(0), lo0, hi0, hi0, done0))
    thr = jnp.where(done_f > 0, vsel_f, hi_f)
    thr_ref[...] = jnp.broadcast_to(thr, (ROW_BLK, 16))


def _score_threshold(sp, ep):
    return pl.pallas_call(
        _score_threshold_body,
        grid=(B // ROW_BLK,),
        in_specs=[
            pl.BlockSpec((ROW_BLK, T), lambda r: (r, 0)),
            pl.BlockSpec((ROW_BLK, T), lambda r: (r, 0)),
        ],
        out_specs=[
            pl.BlockSpec((ROW_BLK, C), lambda r: (r, 0)),
            pl.BlockSpec((ROW_BLK, 16), lambda r: (r, 0)),
        ],
        out_shape=[
            jax.ShapeDtypeStruct((B, C), jnp.int32),
            jax.ShapeDtypeStruct((B, 16), jnp.int32),
        ],
        compiler_params=pltpu.CompilerParams(
            dimension_semantics=("parallel",)),
    )(sp, ep)


ROWS_PER_TILE = 2


def _compact_body(keys_hbm, thr_hbm, ovidx_hbm, okey_hbm,
                  keys_v, cvidx_v, ckey_v, thr_v):
    nc = 2
    wid = lax.axis_index("s") * nc + lax.axis_index("c")
    iota16 = lax.broadcasted_iota(jnp.int32, (16,), 0)
    for rr in range(ROWS_PER_TILE):
        row = wid * ROWS_PER_TILE + rr
        pltpu.sync_copy(keys_hbm.at[row], keys_v)
        pltpu.sync_copy(thr_hbm.at[row], thr_v)
        vthr = thr_v[...]

        @plsc.parallel_loop(0, CPAD // 16, unroll=4)
        def init_step(j):
            cvidx_v[pl.ds(j * 16, 16)] = PAD_VIDX + j * 16 + iota16
            ckey_v[pl.ds(j * 16, 16)] = jnp.zeros((16,), jnp.int32)

        @plsc.parallel_loop(0, C // 16, unroll=8,
                            carry=jnp.zeros((16,), jnp.int32))
        def scan_step(i, off):
            k = keys_v[pl.ds(i * 16, 16)]
            m = k >= vthr
            pc = plsc.cumsum(jnp.where(m, 1, 0))
            pos = off + pc - 1
            em = m & (pos < CPAD)
            plsc.store_scatter(cvidx_v, [pos], i * 16 + iota16, mask=em)
            plsc.store_scatter(ckey_v, [pos], k, mask=em)
            return off + plsc.all_reduce_population_count(m)

        pltpu.sync_copy(cvidx_v, ovidx_hbm.at[row])
        pltpu.sync_copy(ckey_v, okey_hbm.at[row])


@functools.lru_cache(maxsize=1)
def _make_compact():
    return pl.kernel(
        _compact_body,
        out_type=(jax.ShapeDtypeStruct((B, CPAD), jnp.int32),
                  jax.ShapeDtypeStruct((B, CPAD), jnp.int32)),
        mesh=plsc.VectorSubcoreMesh(core_axis_name="c", subcore_axis_name="s"),
        scratch_types=[
            pltpu.VMEM((C,), jnp.int32),
            pltpu.VMEM((CPAD,), jnp.int32),
            pltpu.VMEM((CPAD,), jnp.int32),
            pltpu.VMEM((16,), jnp.int32),
        ],
        compiler_params=pltpu.CompilerParams(needs_layout_passes=False),
    )


RB3 = 4


def _rank_vals_body(vidx_ref, key_ref, vidxT_ref, keyT_ref, rank_ref, vals_ref):
    for rr in range(RB3):
        kj = key_ref[rr]
        vj = vidx_ref[rr]
        ki = keyT_ref[rr]
        vi = vidxT_ref[rr]
        beats = (kj > ki) | ((kj == ki) & (vj < vi))
        rank = jnp.sum(beats.astype(jnp.int32), axis=1, keepdims=True)
        rank_ref[rr] = rank
        g = jnp.minimum(lax.shift_right_logical(vj, 13), 5)
        length = jnp.left_shift(jnp.int32(8), g)
        s = jnp.bitwise_and(vj, T - 1)
        valid = kj >= BITS_MIN
        sf = jnp.where(valid, s.astype(jnp.float32), 0.0)
        ef = jnp.where(valid, (s + length).astype(jnp.float32), float(T))
        scf = jnp.where(valid, lax.bitcast_convert_type(kj, jnp.float32), 0.0)
        vals_ref[rr] = jnp.concatenate([sf, ef, scf], axis=0)


def _rank_vals(cvidx, ckey, cvidx_t, ckey_t):
    return pl.pallas_call(
        _rank_vals_body,
        grid=(B // RB3,),
        in_specs=[
            pl.BlockSpec((RB3, 1, CPAD), lambda r: (r, 0, 0)),
            pl.BlockSpec((RB3, 1, CPAD), lambda r: (r, 0, 0)),
            pl.BlockSpec((RB3, CPAD, 1), lambda r: (r, 0, 0)),
            pl.BlockSpec((RB3, CPAD, 1), lambda r: (r, 0, 0)),
        ],
        out_specs=[
            pl.BlockSpec((RB3, CPAD, 1), lambda r: (r, 0, 0)),
            pl.BlockSpec((RB3, 3, CPAD), lambda r: (r, 0, 0)),
        ],
        out_shape=[
            jax.ShapeDtypeStruct((B, CPAD, 1), jnp.int32),
            jax.ShapeDtypeStruct((B, 3, CPAD), jnp.float32),
        ],
        compiler_params=pltpu.CompilerParams(
            dimension_semantics=("parallel",)),
    )(cvidx[:, None, :], ckey[:, None, :], cvidx_t, ckey_t)


def _permute_body(rank_hbm, vals_hbm, out_hbm, rank_v, vals_v, out_v):
    nc = 2
    wid = lax.axis_index("s") * nc + lax.axis_index("c")
    iota16 = lax.broadcasted_iota(jnp.int32, (16,), 0)
    del iota16
    for rr in range(ROWS_PER_TILE):
        row = wid * ROWS_PER_TILE + rr
        pltpu.sync_copy(rank_hbm.at[row], rank_v)
        pltpu.sync_copy(vals_hbm.at[row], vals_v)

        @plsc.parallel_loop(0, CPAD // 16, unroll=4)
        def step(i):
            r3 = rank_v[pl.ds(i * 16, 16)] * 3
            for c in range(3):
                x = vals_v[c, pl.ds(i * 16, 16)]
                plsc.store_scatter(out_v, [r3 + c], x)

        pltpu.sync_copy(out_v, out_hbm.at[row])


@functools.lru_cache(maxsize=1)
def _make_permute():
    return pl.kernel(
        _permute_body,
        out_type=jax.ShapeDtypeStruct((B, CPAD * 3), jnp.float32),
        mesh=plsc.VectorSubcoreMesh(core_axis_name="c", subcore_axis_name="s"),
        scratch_types=[
            pltpu.VMEM((CPAD,), jnp.int32),
            pltpu.VMEM((3, CPAD), jnp.float32),
            pltpu.VMEM((CPAD * 3,), jnp.float32),
        ],
        compiler_params=pltpu.CompilerParams(needs_layout_passes=False),
    )


def kernel(start_probs, end_probs):
    sp = start_probs[:, 0, :]
    ep = end_probs[:, 0, :]
    keys, thr = _score_threshold(sp, ep)
    cvidx, ckey = _make_compact()(keys, thr)
    rank, vals = _rank_vals(cvidx, ckey, cvidx[:, :, None], ckey[:, :, None])
    out = _make_permute()(rank[:, :, 0], vals)
    return out.reshape(B, CPAD, 3)[:, :K, :]


# speedup vs baseline: 1.3119x; 1.3063x over previous
import functools

import jax
import jax.numpy as jnp
from jax import lax
from jax.experimental import pallas as pl
from jax.experimental.pallas import tpu as pltpu
from jax.experimental.pallas import tpu_sc as plsc

LENGTHS = (8, 16, 32, 64, 128, 256)
MIN_SCORE = 0.1
K = 1000
B, T = 64, 8192
G = len(LENGTHS)
C = G * T
BITS_MIN = 1036831949
BITS_ONE = 1065353216
ROW_BLK = 8
CPAD = 1024
PAD_VIDX = 1 << 20


def _score_threshold_body(sp_ref, ep_ref, keys_ref, thr_ref):
    sp = sp_ref[...]
    ep = ep_ref[...]
    parts = []
    for g, length in enumerate(LENGTHS):
        w = T - length + 1
        sc = jnp.sqrt(sp[:, :w] * ep[:, length - 1:])
        kb = lax.bitcast_convert_type(sc, jnp.int32)
        kb = jnp.where(sc >= MIN_SCORE, kb, 0)
        parts.append(kb)
        parts.append(jnp.zeros((ROW_BLK, length - 1), jnp.int32))
    keys = jnp.concatenate(parts, axis=1)
    keys_ref[...] = keys

    def bs_cond(st):
        t, lo, hi, vsel, done = st
        return jnp.logical_and(t < 26, jnp.sum(done) < ROW_BLK)

    def bs_step(st):
        t, lo, hi, vsel, done = st
        mid = (lo + hi) >> 1
        cnt = jnp.sum((keys > mid).astype(jnp.int32), axis=1, keepdims=True)
        small = cnt < K
        doneb = done > 0
        in_win = (cnt >= K) & (cnt <= CPAD) & jnp.logical_not(doneb)
        frozen = doneb | in_win
        lo = jnp.where(frozen, lo, jnp.where(small, lo, mid + 1))
        hi = jnp.where(frozen, hi, jnp.where(small, mid, hi))
        vsel = jnp.where(in_win, mid + 1, vsel)
        conv = (lo >= hi) & jnp.logical_not(frozen)
        vsel = jnp.where(conv, hi, vsel)
        done = jnp.where(in_win | conv, 1, done)
        return t + 1, lo, hi, vsel, done

    lo0 = jnp.full((ROW_BLK, 1), BITS_MIN - 1, jnp.int32)
    hi0 = jnp.full((ROW_BLK, 1), BITS_ONE, jnp.int32)
    done0 = jnp.zeros((ROW_BLK, 1), jnp.int32)
    _, _, hi_f, vsel_f, done_f = lax.while_loop(
        bs_cond, bs_step, (jnp.int32(0), lo0, hi0, hi0, done0))
    thr = jnp.where(done_f > 0, vsel_f, hi_f)
    thr_ref[...] = jnp.broadcast_to(thr, (ROW_BLK, 16))


def _score_threshold(sp, ep):
    return pl.pallas_call(
        _score_threshold_body,
        grid=(B // ROW_BLK,),
        in_specs=[
            pl.BlockSpec((ROW_BLK, T), lambda r: (r, 0)),
            pl.BlockSpec((ROW_BLK, T), lambda r: (r, 0)),
        ],
        out_specs=[
            pl.BlockSpec((ROW_BLK, C), lambda r: (r, 0)),
            pl.BlockSpec((ROW_BLK, 16), lambda r: (r, 0)),
        ],
        out_shape=[
            jax.ShapeDtypeStruct((B, C), jnp.int32),
            jax.ShapeDtypeStruct((B, 16), jnp.int32),
        ],
        compiler_params=pltpu.CompilerParams(
            dimension_semantics=("parallel",)),
    )(sp, ep)


ROWS_PER_TILE = 2


def _compact_body(keys_hbm, thr_hbm, ovidx_hbm, okey_hbm,
                  keys_v, cvidx_v, ckey_v, thr_v):
    nc = 2
    wid = lax.axis_index("s") * nc + lax.axis_index("c")
    iota16 = lax.broadcasted_iota(jnp.int32, (16,), 0)
    for rr in range(ROWS_PER_TILE):
        row = wid * ROWS_PER_TILE + rr
        pltpu.sync_copy(keys_hbm.at[row], keys_v)
        pltpu.sync_copy(thr_hbm.at[row], thr_v)
        vthr = thr_v[...]

        @plsc.parallel_loop(0, CPAD // 16, unroll=4)
        def init_step(j):
            cvidx_v[pl.ds(j * 16, 16)] = PAD_VIDX + j * 16 + iota16
            ckey_v[pl.ds(j * 16, 16)] = jnp.zeros((16,), jnp.int32)

        @plsc.parallel_loop(0, C // 16, unroll=8,
                            carry=jnp.zeros((16,), jnp.int32))
        def scan_step(i, off):
            k = keys_v[pl.ds(i * 16, 16)]
            m = k >= vthr
            pc = plsc.cumsum(jnp.where(m, 1, 0))
            pos = off + pc - 1
            em = m & (pos < CPAD)
            plsc.store_scatter(cvidx_v, [pos], i * 16 + iota16, mask=em)
            plsc.store_scatter(ckey_v, [pos], k, mask=em)
            return off + plsc.all_reduce_population_count(m)

        pltpu.sync_copy(cvidx_v, ovidx_hbm.at[row])
        pltpu.sync_copy(ckey_v, okey_hbm.at[row])


@functools.lru_cache(maxsize=1)
def _make_compact():
    return pl.kernel(
        _compact_body,
        out_type=(jax.ShapeDtypeStruct((B, CPAD), jnp.int32),
                  jax.ShapeDtypeStruct((B, CPAD), jnp.int32)),
        mesh=plsc.VectorSubcoreMesh(core_axis_name="c", subcore_axis_name="s"),
        scratch_types=[
            pltpu.VMEM((C,), jnp.int32),
            pltpu.VMEM((CPAD,), jnp.int32),
            pltpu.VMEM((CPAD,), jnp.int32),
            pltpu.VMEM((16,), jnp.int32),
        ],
        compiler_params=pltpu.CompilerParams(needs_layout_passes=False),
    )


RB3 = 4


def _rank_vals_body(vidx_ref, key_ref, keyT_ref, rank_ref, vals_ref):
    tri = (lax.broadcasted_iota(jnp.int32, (CPAD, CPAD), 1)
           < lax.broadcasted_iota(jnp.int32, (CPAD, CPAD), 0))
    for rr in range(RB3):
        kj = key_ref[rr]
        vj = vidx_ref[rr]
        ki = keyT_ref[rr]
        beats = (kj > ki) | ((kj == ki) & tri)
        rank = jnp.sum(beats.astype(jnp.int32), axis=1, keepdims=True)
        rank_ref[rr] = rank
        g = jnp.minimum(lax.shift_right_logical(vj, 13), 5)
        length = jnp.left_shift(jnp.int32(8), g)
        s = jnp.bitwise_and(vj, T - 1)
        valid = kj >= BITS_MIN
        sf = jnp.where(valid, s.astype(jnp.float32), 0.0)
        ef = jnp.where(valid, (s + length).astype(jnp.float32), float(T))
        scf = jnp.where(valid, lax.bitcast_convert_type(kj, jnp.float32), 0.0)
        vals_ref[rr] = jnp.concatenate([sf, ef, scf], axis=0)


def _rank_vals(cvidx, ckey, ckey_t):
    return pl.pallas_call(
        _rank_vals_body,
        grid=(B // RB3,),
        in_specs=[
            pl.BlockSpec((RB3, 1, CPAD), lambda r: (r, 0, 0)),
            pl.BlockSpec((RB3, 1, CPAD), lambda r: (r, 0, 0)),
            pl.BlockSpec((RB3, CPAD, 1), lambda r: (r, 0, 0)),
        ],
        out_specs=[
            pl.BlockSpec((RB3, CPAD, 1), lambda r: (r, 0, 0)),
            pl.BlockSpec((RB3, 3, CPAD), lambda r: (r, 0, 0)),
        ],
        out_shape=[
            jax.ShapeDtypeStruct((B, CPAD, 1), jnp.int32),
            jax.ShapeDtypeStruct((B, 3, CPAD), jnp.float32),
        ],
        compiler_params=pltpu.CompilerParams(
            dimension_semantics=("parallel",)),
    )(cvidx[:, None, :], ckey[:, None, :], ckey_t)


def _permute_body(rank_hbm, vals_hbm, out_hbm, rank_v, vals_v, out_v):
    nc = 2
    wid = lax.axis_index("s") * nc + lax.axis_index("c")
    iota16 = lax.broadcasted_iota(jnp.int32, (16,), 0)
    del iota16
    for rr in range(ROWS_PER_TILE):
        row = wid * ROWS_PER_TILE + rr
        pltpu.sync_copy(rank_hbm.at[row], rank_v)
        pltpu.sync_copy(vals_hbm.at[row], vals_v)

        @plsc.parallel_loop(0, CPAD // 16, unroll=4)
        def step(i):
            r = rank_v[pl.ds(i * 16, 16)]
            r3 = r * 3
            em = r < K
            for c in range(3):
                x = vals_v[c, pl.ds(i * 16, 16)]
                plsc.store_scatter(out_v, [r3 + c], x, mask=em)

        pltpu.sync_copy(out_v, out_hbm.at[row])


@functools.lru_cache(maxsize=1)
def _make_permute():
    return pl.kernel(
        _permute_body,
        out_type=jax.ShapeDtypeStruct((B, K * 3), jnp.float32),
        mesh=plsc.VectorSubcoreMesh(core_axis_name="c", subcore_axis_name="s"),
        scratch_types=[
            pltpu.VMEM((CPAD,), jnp.int32),
            pltpu.VMEM((3, CPAD), jnp.float32),
            pltpu.VMEM((K * 3,), jnp.float32),
        ],
        compiler_params=pltpu.CompilerParams(needs_layout_passes=False),
    )


def kernel(start_probs, end_probs):
    sp = start_probs[:, 0, :]
    ep = end_probs[:, 0, :]
    keys, thr = _score_threshold(sp, ep)
    cvidx, ckey = _make_compact()(keys, thr)
    rank, vals = _rank_vals(cvidx, ckey, ckey[:, :, None])
    out = _make_permute()(rank[:, :, 0], vals)
    return out.reshape(B, K, 3)
